# Initial kernel scaffold; baseline (speedup 1.0000x reference)
import jax
import jax.numpy as jnp
from jax.experimental import pallas as pl


def kernel(x, edge_index, batch, W1, att_src1, att_dst1, bias1, W2, att_src2, att_dst2, bias2, bn1_gamma, bn1_beta, bn2_gamma, bn2_beta, fc_W, fc_b):
    raise NotImplementedError("write your pallas kernel here")


# speedup vs baseline: 7.5505x; 7.5505x over previous
import functools

import jax
import jax.numpy as jnp
from jax import lax
from jax.experimental import pallas as pl
from jax.experimental.pallas import tpu as pltpu
from jax.experimental.pallas import tpu_sc as plsc

N = 10000
E = 160000
IN = 256
HID = 256
HEADS = 4
OUT = 128
G = 64

NC = 2
NS = 16
L = 16

EPT = E // NS
SUB = 400
NSUB = EPT // SUB
SCAN = 2000
NSCAN = EPT // SCAN
NCHUNK = 20
CPS = NCHUNK // NC
CN = 512
CROWS = CN * HEADS
RPT = CROWS // NS
WCAP = 1024
DENR = 64
NPT = N // NS
KV = IN // L

import math
_BN_C = 1.0 / math.sqrt(1.0 + 1e-5)


B1 = 64
WCAP = 1024
OCAP = 1024
NPC = 32
AGR = NPC * HEADS


def _sc_body(sad_hbm, src_hbm, dst_hbm, x_hbm, agg_hbm,
             wlsp_src, wlsp_dst, cnts_sp,
             sblk, dblk, wl_src, wl_dstl, cntob, cntib,
             in_src, in_dst, own_src, own_dstl, own_ex,
             idx32, adst_own, gb_s, s64,
             den_own, den_red, agg_own, xrows, cbuf, sem):
  c = lax.axis_index("c")
  t = lax.axis_index("s")
  ebase = t * EPT
  iot = lax.iota(jnp.int32, L)
  zf = jnp.zeros((L,), jnp.float32)
  zi = jnp.zeros((L,), jnp.int32)

  for p in range(CPS):
    chunk = c * CPS + p
    lo = chunk * CN
    obase = lo + t * NPC

    def _zagg(r, _):
      for k in range(KV):
        agg_own[r, pl.ds(k * L, L)] = zf
      return 0
    lax.fori_loop(0, AGR, _zagg, 0)

    def _zden(r, _):
      den_own[pl.ds(r * L, L)] = zf
      return 0
    lax.fori_loop(0, NPC * HEADS * L // L, _zden, 0)

    def _scan(blk, cnt):
      off = ebase + blk * SCAN
      pltpu.sync_copy(src_hbm.at[pl.ds(off, SCAN)], sblk)
      pltpu.sync_copy(dst_hbm.at[pl.ds(off, SCAN)], dblk)

      def _cgrp(g, cnt):
        o = g * L
        dv = dblk[pl.ds(o, L)]
        m = (dv >= lo) & (dv < lo + CN)
        cs = plsc.cumsum(jnp.where(m, 1, 0))
        pos = cs + (cnt - 1)
        plsc.store_scatter(wl_src, [pos], sblk[pl.ds(o, L)], mask=m)
        plsc.store_scatter(wl_dstl, [pos], dv - lo, mask=m)
        return cnt + jnp.max(cs)
      return lax.fori_loop(0, SCAN // L, _cgrp, cnt)
    cnt = lax.fori_loop(0, NSCAN, _scan, jnp.int32(0))

    cntob[pl.ds(0, L)] = jnp.full((L,), 0, jnp.int32) + cnt
    pltpu.sync_copy(wl_src, wlsp_src.at[pl.ds(t * WCAP, WCAP)])
    pltpu.sync_copy(wl_dstl, wlsp_dst.at[pl.ds(t * WCAP, WCAP)])
    pltpu.sync_copy(cntob, cnts_sp.at[pl.ds(t * 128, 128)])
    plsc.subcore_barrier()

    def _pull(t2, ocnt):
      pltpu.sync_copy(wlsp_src.at[pl.ds(t2 * WCAP, WCAP)], in_src)
      pltpu.sync_copy(wlsp_dst.at[pl.ds(t2 * WCAP, WCAP)], in_dst)
      pltpu.sync_copy(cnts_sp.at[pl.ds(t2 * 128, 128)], cntib)
      cv = cntib[pl.ds(0, L)]
      cnt2 = cv[0]
      ng2 = (cnt2 + (L - 1)) // L

      def _fgrp(g, ocnt):
        o = g * L
        dv = in_dst[pl.ds(o, L)]
        m = ((dv >> 5) == t) & ((iot + o) < cnt2)
        cs = plsc.cumsum(jnp.where(m, 1, 0))
        pos = cs + (ocnt - 1)
        plsc.store_scatter(own_src, [pos], in_src[pl.ds(o, L)], mask=m)
        plsc.store_scatter(own_dstl, [pos], dv & (NPC - 1), mask=m)
        return ocnt + jnp.max(cs)
      return lax.fori_loop(0, ng2, _fgrp, ocnt)
    ocnt = lax.fori_loop(0, NS, _pull, jnp.int32(0))

    def _pad(k, _):
      own_src[pl.ds(ocnt + k * L, L)] = zi
      own_dstl[pl.ds(ocnt + k * L, L)] = zi
      return 0
    lax.fori_loop(0, B1 // L, _pad, 0)

    idx32[pl.ds(0, L)] = jnp.minimum(iot + obase, N - 1)
    idx32[pl.ds(L, L)] = jnp.minimum(iot + (obase + L), N - 1)
    pltpu.async_copy(sad_hbm.at[idx32], adst_own, sem).wait()

    nb1 = (ocnt + (B1 - 1)) // B1

    def _w1(b, _):
      o = b * B1

      def _prep(k, _):
        s64[pl.ds(k * L, L)] = own_src[pl.ds(o + k * L, L)]
        return 0
      lax.fori_loop(0, B1 // L, _prep, 0)
      pltpu.async_copy(sad_hbm.at[s64], gb_s, sem).wait()

      def _grp(g, _):
        rid = iot + g * L
        validm = (rid + o) < ocnt
        dl = own_dstl[pl.ds(o + g * L, L)]
        for h in range(HEADS):
          fh = jnp.full((L,), h, jnp.int32)
          asr = plsc.load_gather(gb_s, [rid, fh])
          ads = plsc.load_gather(adst_own, [dl, jnp.full((L,), h + HEADS, jnp.int32)])
          a = asr + ads
          a = jnp.maximum(a, a * 0.2)
          exv = jnp.where(validm, jnp.exp(a), 0.0)
          own_ex[pl.ds(h * OCAP + o + g * L, L)] = exv
          plsc.addupdate_scatter(den_own, [dl * (HEADS * L) + h * L + iot], exv)
        return 0
      lax.fori_loop(0, B1 // L, _grp, 0)
      return 0
    lax.fori_loop(0, nb1, _w1, 0)

    def _dred(r, _):
      dsum = jnp.sum(den_own[pl.ds(r * L, L)])
      den_red[r, :] = jnp.full((L,), 0.0, jnp.float32) + dsum
      return 0
    lax.fori_loop(0, NPC * HEADS, _dred, 0)

    nw = (ocnt + (L - 1)) // L

    def _proc(j, _):
      o = j * L
      sv = own_src[pl.ds(o, L)]
      pltpu.async_copy(x_hbm.at[sv], xrows, sem).wait()
      dl = own_dstl[pl.ds(o, L)]
      zc = jnp.full((L,), 0, jnp.int32)
      for h in range(HEADS):
        exv = own_ex[pl.ds(h * OCAP + o, L)]
        dnv = plsc.load_gather(den_red, [dl * HEADS + h, zc])
        cbuf[h, :] = exv / jnp.maximum(dnv, 1e-30)

      def _pere(e, _):
        row = [xrows[e, pl.ds(k * L, L)] for k in range(KV)]
        ev = jnp.full((L,), 0, jnp.int32) + e
        dle = plsc.load_gather(own_dstl, [ev + o])
        for h in range(HEADS):
          ch = plsc.load_gather(cbuf, [jnp.full((L,), h, jnp.int32), ev])
          r4 = dle * HEADS + h
          for k in range(KV):
            cur = plsc.load_gather(agg_own, [r4, iot + k * L])
            plsc.store_scatter(agg_own, [r4, iot + k * L], cur + row[k] * ch)
        return 0
      lax.fori_loop(0, L, _pere, 0)
      return 0
    lax.fori_loop(0, nw, _proc, jnp.int32(0))

    pltpu.sync_copy(agg_own, agg_hbm.at[pl.ds(lo * HEADS + t * AGR, AGR), :])
    plsc.subcore_barrier()


_sc_edge = pl.kernel(
    _sc_body,
    out_type=jax.ShapeDtypeStruct((NCHUNK * CN * HEADS, IN), jnp.float32),
    mesh=plsc.VectorSubcoreMesh(core_axis_name="c", subcore_axis_name="s",
                                num_cores=NC, num_subcores=NS),
    compiler_params=pltpu.CompilerParams(needs_layout_passes=False),
    scratch_types=[
        pltpu.VMEM_SHARED((NS * WCAP,), jnp.int32),
        pltpu.VMEM_SHARED((NS * WCAP,), jnp.int32),
        pltpu.VMEM_SHARED((NS * 128,), jnp.int32),
        pltpu.VMEM((SCAN,), jnp.int32),
        pltpu.VMEM((SCAN,), jnp.int32),
        pltpu.VMEM((WCAP,), jnp.int32),
        pltpu.VMEM((WCAP,), jnp.int32),
        pltpu.VMEM((128,), jnp.int32),
        pltpu.VMEM((128,), jnp.int32),
        pltpu.VMEM((WCAP,), jnp.int32),
        pltpu.VMEM((WCAP,), jnp.int32),
        pltpu.VMEM((OCAP + B1,), jnp.int32),
        pltpu.VMEM((OCAP + B1,), jnp.int32),
        pltpu.VMEM((HEADS * OCAP + B1,), jnp.float32),
        pltpu.VMEM((NPC,), jnp.int32),
        pltpu.VMEM((NPC, 128), jnp.float32),
        pltpu.VMEM((B1, 128), jnp.float32),
        pltpu.VMEM((B1,), jnp.int32),
        pltpu.VMEM((NPC * HEADS * L,), jnp.float32),
        pltpu.VMEM((NPC * HEADS, L), jnp.float32),
        pltpu.VMEM((AGR, IN), jnp.float32),
        pltpu.VMEM((L, IN), jnp.float32),
        pltpu.VMEM((HEADS, L), jnp.float32),
        pltpu.SemaphoreType.DMA,
    ],
)


def _alpha_vecs(W_ref, as_ref, ad_ref):
  vs, vd = [], []
  for h in range(HEADS):
    Wh = W_ref[:, h * HID:(h + 1) * HID]
    sb = jnp.broadcast_to(as_ref[h:h + 1, :], (IN, HID))
    db = jnp.broadcast_to(ad_ref[h:h + 1, :], (IN, HID))
    vs.append(jnp.sum(Wh * sb, axis=1, keepdims=True))
    vd.append(jnp.sum(Wh * db, axis=1, keepdims=True))
  return (jnp.concatenate(vs, axis=1), jnp.concatenate(vd, axis=1))


def _sad_body(x_ref, W_ref, as_ref, ad_ref, o_ref):
  Vs, Vd = _alpha_vecs(W_ref, as_ref, ad_ref)
  xb = x_ref[...]
  a_s = jnp.dot(xb, Vs, preferred_element_type=jnp.float32)
  a_d = jnp.dot(xb, Vd, preferred_element_type=jnp.float32)
  pad = jnp.zeros((xb.shape[0], 120), jnp.float32)
  o_ref[...] = jnp.concatenate([a_s, a_d, pad], axis=1)


def _sad_call(x, W, att_s, att_d):
  return pl.pallas_call(
      _sad_body,
      grid=(10,),
      in_specs=[
          pl.BlockSpec((1000, IN), lambda i: (i, 0)),
          pl.BlockSpec((IN, HEADS * HID), lambda i: (0, 0)),
          pl.BlockSpec((HEADS, HID), lambda i: (0, 0)),
          pl.BlockSpec((HEADS, HID), lambda i: (0, 0)),
      ],
      out_specs=pl.BlockSpec((1000, 128), lambda i: (i, 0)),
      out_shape=jax.ShapeDtypeStruct((N, 128), jnp.float32),
  )(x, W, att_s, att_d)


def _dense_body(A_ref, Ws_ref, b_ref, g_ref, be_ref, W2_ref, as_ref, ad_ref,
                h_ref, sad_ref):
  acc = jnp.dot(A_ref[...], Ws_ref[...], preferred_element_type=jnp.float32)
  y = (acc * 0.25 + b_ref[...]) * (g_ref[...] * _BN_C) + be_ref[...]
  hv = jnp.maximum(y, 0.0)
  h_ref[...] = hv
  Vs, Vd = _alpha_vecs(W2_ref, as_ref, ad_ref)
  a_s = jnp.dot(hv, Vs, preferred_element_type=jnp.float32)
  a_d = jnp.dot(hv, Vd, preferred_element_type=jnp.float32)
  pad = jnp.zeros((hv.shape[0], 120), jnp.float32)
  sad_ref[...] = jnp.concatenate([a_s, a_d, pad], axis=1)


def _dense_call(A, Ws, b, g, be, W2, att_s2, att_d2):
  return pl.pallas_call(
      _dense_body,
      grid=(10,),
      in_specs=[
          pl.BlockSpec((1000, HEADS * IN), lambda i: (i, 0)),
          pl.BlockSpec((HEADS * IN, HID), lambda i: (0, 0)),
          pl.BlockSpec((1, HID), lambda i: (0, 0)),
          pl.BlockSpec((1, HID), lambda i: (0, 0)),
          pl.BlockSpec((1, HID), lambda i: (0, 0)),
          pl.BlockSpec((HID, HEADS * HID), lambda i: (0, 0)),
          pl.BlockSpec((HEADS, HID), lambda i: (0, 0)),
          pl.BlockSpec((HEADS, HID), lambda i: (0, 0)),
      ],
      out_specs=[
          pl.BlockSpec((1000, HID), lambda i: (i, 0)),
          pl.BlockSpec((1000, 128), lambda i: (i, 0)),
      ],
      out_shape=[
          jax.ShapeDtypeStruct((N, HID), jnp.float32),
          jax.ShapeDtypeStruct((N, 128), jnp.float32),
      ],
  )(A, Ws, b, g, be, W2, att_s2, att_d2)


def _final_body(A_ref, Ws_ref, b_ref, g_ref, be_ref, bat_ref, fw_ref, fb_ref,
                o_ref, sums, counts):
  i = pl.program_id(0)
  acc = jnp.dot(A_ref[...], Ws_ref[...], preferred_element_type=jnp.float32)
  y = (acc * 0.25 + b_ref[...]) * (g_ref[...] * _BN_C) + be_ref[...]
  hv = jnp.maximum(y, 0.0)

  bvec = bat_ref[0, 0, :]
  gid = lax.broadcasted_iota(jnp.int32, (G, 1000), 0)
  onehot = jnp.where(gid == bvec[None, :], 1.0, 0.0).astype(jnp.float32)

  @pl.when(i == 0)
  def _init():
    sums[...] = jnp.zeros_like(sums)
    counts[...] = jnp.zeros_like(counts)

  sums[...] += jnp.dot(onehot, hv, preferred_element_type=jnp.float32)
  counts[...] += jnp.broadcast_to(
      jnp.sum(onehot, axis=1, keepdims=True), (G, HID))

  @pl.when(i == 9)
  def _fin():
    pooled = sums[...] / jnp.maximum(counts[...], 1.0)
    o_ref[...] = (jnp.dot(pooled, fw_ref[...], preferred_element_type=jnp.float32)
                  + fb_ref[...])


def _final_call(A, Ws, b, g, be, batch3, fw, fb):
  return pl.pallas_call(
      _final_body,
      grid=(10,),
      in_specs=[
          pl.BlockSpec((1000, HEADS * HID), lambda i: (i, 0)),
          pl.BlockSpec((HEADS * HID, HID), lambda i: (0, 0)),
          pl.BlockSpec((1, HID), lambda i: (0, 0)),
          pl.BlockSpec((1, HID), lambda i: (0, 0)),
          pl.BlockSpec((1, HID), lambda i: (0, 0)),
          pl.BlockSpec((1, 1, 1000), lambda i: (i, 0, 0)),
          pl.BlockSpec((HID, OUT), lambda i: (0, 0)),
          pl.BlockSpec((1, OUT), lambda i: (0, 0)),
      ],
      out_specs=pl.BlockSpec((G, OUT), lambda i: (0, 0)),
      out_shape=jax.ShapeDtypeStruct((G, OUT), jnp.float32),
      scratch_shapes=[
          pltpu.VMEM((G, HID), jnp.float32),
          pltpu.VMEM((G, HID), jnp.float32),
      ],
  )(A, Ws, b, g, be, batch3, fw, fb)


def kernel(x, edge_index, batch, W1, att_src1, att_dst1, bias1,
           W2, att_src2, att_dst2, bias2,
           bn1_gamma, bn1_beta, bn2_gamma, bn2_beta, fc_W, fc_b):
  src = edge_index[0].astype(jnp.int32)
  dst = edge_index[1].astype(jnp.int32)
  batch3 = batch.astype(jnp.int32).reshape(10, 1, 1000)

  Ws1 = W1.reshape(IN, HEADS, HID).transpose(1, 0, 2).reshape(HEADS * IN, HID)
  Ws2 = W2.reshape(HID, HEADS, HID).transpose(1, 0, 2).reshape(HEADS * HID, HID)
  b1 = bias1.reshape(1, HID)
  b2 = bias2.reshape(1, HID)
  g1 = bn1_gamma.reshape(1, HID)
  g2 = bn2_gamma.reshape(1, HID)
  be1 = bn1_beta.reshape(1, HID)
  be2 = bn2_beta.reshape(1, HID)
  fbr = fc_b.reshape(1, OUT)

  sad1 = _sad_call(x, W1, att_src1, att_dst1)
  agg1 = _sc_edge(sad1, src, dst, x)
  h1, sad2 = _dense_call(agg1.reshape(-1, HEADS * IN), Ws1, b1, g1, be1,
                         W2, att_src2, att_dst2)
  agg2 = _sc_edge(sad2, src, dst, h1)
  return _final_call(agg2.reshape(-1, HEADS * HID), Ws2, b2, g2, be2,
                     batch3, fc_W, fbr)


# speedup vs baseline: 14.2133x; 1.8824x over previous
import functools

import jax
import jax.numpy as jnp
from jax import lax
from jax.experimental import pallas as pl
from jax.experimental.pallas import tpu as pltpu
from jax.experimental.pallas import tpu_sc as plsc

N = 10000
E = 160000
IN = 256
HID = 256
HEADS = 4
OUT = 128
G = 64

NC = 2
NS = 16
L = 16

EPT = E // NS
SUB = 400
NSUB = EPT // SUB
SCAN = 2000
NSCAN = EPT // SCAN
NCHUNK = 20
CPS = NCHUNK // NC
CN = 512
CROWS = CN * HEADS
RPT = CROWS // NS
WCAP = 1024
DENR = 64
NPT = N // NS
KV = IN // L

import math
_BN_C = 1.0 / math.sqrt(1.0 + 1e-5)


B1 = 64
WCAP = 1024
OCAP = 1024
NPC = 32
AGR = NPC * HEADS


def _sc_body(sad_hbm, src_hbm, dst_hbm, x_hbm, agg_hbm,
             wlsp_src, wlsp_dst, cnts_sp,
             sblk, dblk, wl_src, wl_dstl, cntob, cntib,
             in_src, in_dst, own_src, own_dstl, own_ex,
             idx32, adst_own, gb_s, s64,
             den_own, den_red, agg_own, xrows, xrows2, cbuf, sem, sem2):
  c = lax.axis_index("c")
  t = lax.axis_index("s")
  ebase = t * EPT
  iot = lax.iota(jnp.int32, L)
  zf = jnp.zeros((L,), jnp.float32)
  zi = jnp.zeros((L,), jnp.int32)

  def _pass(p, _):
    chunk = c * CPS + p
    lo = chunk * CN
    obase = lo + t * NPC

    def _zagg(r, _):
      for k in range(KV):
        agg_own[r, pl.ds(k * L, L)] = zf
      return 0
    lax.fori_loop(0, AGR, _zagg, 0)

    def _zden(r, _):
      den_own[pl.ds(r * L, L)] = zf
      return 0
    lax.fori_loop(0, NPC * HEADS * L // L, _zden, 0)

    def _scan(blk, cnt):
      off = ebase + blk * SCAN
      pltpu.sync_copy(src_hbm.at[pl.ds(off, SCAN)], sblk)
      pltpu.sync_copy(dst_hbm.at[pl.ds(off, SCAN)], dblk)

      def _cgrp(g, cnt):
        o = g * L
        dv = dblk[pl.ds(o, L)]
        m = (dv >= lo) & (dv < lo + CN)
        cs = plsc.cumsum(jnp.where(m, 1, 0))
        pos = cs + (cnt - 1)
        plsc.store_scatter(wl_src, [pos], sblk[pl.ds(o, L)], mask=m)
        plsc.store_scatter(wl_dstl, [pos], dv - lo, mask=m)
        return cnt + jnp.max(cs)
      return lax.fori_loop(0, SCAN // L, _cgrp, cnt)
    cnt = lax.fori_loop(0, NSCAN, _scan, jnp.int32(0))

    cntob[pl.ds(0, L)] = jnp.full((L,), 0, jnp.int32) + cnt
    pltpu.sync_copy(wl_src, wlsp_src.at[pl.ds(t * WCAP, WCAP)])
    pltpu.sync_copy(wl_dstl, wlsp_dst.at[pl.ds(t * WCAP, WCAP)])
    pltpu.sync_copy(cntob, cnts_sp.at[pl.ds(t * 128, 128)])
    plsc.subcore_barrier()

    pltpu.sync_copy(wlsp_src, in_src)
    pltpu.sync_copy(wlsp_dst, in_dst)
    pltpu.sync_copy(cnts_sp, cntib)

    def _pull(t2, ocnt):
      cv = cntib[pl.ds(t2 * 128, L)]
      cnt2 = cv[0]
      ng2 = (cnt2 + (L - 1)) // L
      tb = t2 * WCAP

      def _fgrp(g, ocnt):
        o = tb + g * L
        dv = in_dst[pl.ds(o, L)]
        m = ((dv >> 5) == t) & ((iot + g * L) < cnt2)
        cs = plsc.cumsum(jnp.where(m, 1, 0))
        pos = cs + (ocnt - 1)
        plsc.store_scatter(own_src, [pos], in_src[pl.ds(o, L)], mask=m)
        plsc.store_scatter(own_dstl, [pos], dv & (NPC - 1), mask=m)
        return ocnt + jnp.max(cs)
      return lax.fori_loop(0, ng2, _fgrp, ocnt)
    ocnt = lax.fori_loop(0, NS, _pull, jnp.int32(0))

    def _pad(k, _):
      own_src[pl.ds(ocnt + k * L, L)] = zi
      own_dstl[pl.ds(ocnt + k * L, L)] = zi
      return 0
    lax.fori_loop(0, B1 // L, _pad, 0)

    idx32[pl.ds(0, L)] = jnp.minimum(iot + obase, N - 1)
    idx32[pl.ds(L, L)] = jnp.minimum(iot + (obase + L), N - 1)
    pltpu.async_copy(sad_hbm.at[idx32], adst_own, sem).wait()

    nb1 = (ocnt + (B1 - 1)) // B1

    def _w1(b, _):
      o = b * B1

      def _prep(k, _):
        s64[pl.ds(k * L, L)] = own_src[pl.ds(o + k * L, L)]
        return 0
      lax.fori_loop(0, B1 // L, _prep, 0)
      pltpu.async_copy(sad_hbm.at[s64], gb_s, sem).wait()

      def _grp(g, _):
        rid = iot + g * L
        validm = (rid + o) < ocnt
        dl = own_dstl[pl.ds(o + g * L, L)]
        for h in range(HEADS):
          fh = jnp.full((L,), h, jnp.int32)
          asr = plsc.load_gather(gb_s, [rid, fh])
          ads = plsc.load_gather(adst_own, [dl, jnp.full((L,), h + HEADS, jnp.int32)])
          a = asr + ads
          a = jnp.maximum(a, a * 0.2)
          exv = jnp.where(validm, jnp.exp(a), 0.0)
          own_ex[pl.ds(h * OCAP + o + g * L, L)] = exv
          plsc.addupdate_scatter(den_own, [dl * (HEADS * L) + h * L + iot], exv)
        return 0
      lax.fori_loop(0, B1 // L, _grp, 0)
      return 0
    lax.fori_loop(0, nb1, _w1, 0)

    def _dred(r, _):
      dsum = jnp.sum(den_own[pl.ds(r * L, L)])
      den_red[r, :] = jnp.full((L,), 0.0, jnp.float32) + dsum
      return 0
    lax.fori_loop(0, NPC * HEADS, _dred, 0)

    npair = (ocnt + (2 * L - 1)) // (2 * L)

    def _half(o, xr):
      dl = own_dstl[pl.ds(o, L)]
      zc = jnp.full((L,), 0, jnp.int32)
      for h in range(HEADS):
        exv = own_ex[pl.ds(h * OCAP + o, L)]
        dnv = plsc.load_gather(den_red, [dl * HEADS + h, zc])
        cbuf[h, :] = exv / jnp.maximum(dnv, 1e-30)

      def _pere(e, _):
        row = [xr[e, pl.ds(k * L, L)] for k in range(KV)]
        ev = jnp.full((L,), 0, jnp.int32) + e
        dle = plsc.load_gather(own_dstl, [ev + o])
        for h in range(HEADS):
          ch = plsc.load_gather(cbuf, [jnp.full((L,), h, jnp.int32), ev])
          r4 = dle * HEADS + h
          for k in range(KV):
            plsc.addupdate_scatter(agg_own, [r4, iot + k * L], row[k] * ch)
        return 0
      lax.fori_loop(0, L, _pere, 0)

    cp0 = pltpu.async_copy(x_hbm.at[own_src[pl.ds(0, L)]], xrows, sem)

    def _proc(j, _):
      o = 2 * j * L
      cpb = pltpu.async_copy(x_hbm.at[own_src[pl.ds(o + L, L)]], xrows2, sem2)
      pltpu.make_async_copy(x_hbm.at[own_src[pl.ds(o, L)]], xrows, sem).wait()
      _half(o, xrows)
      cpa = pltpu.async_copy(x_hbm.at[own_src[pl.ds(o + 2 * L, L)]], xrows, sem)
      pltpu.make_async_copy(x_hbm.at[own_src[pl.ds(o + L, L)]], xrows2, sem2).wait()
      _half(o + L, xrows2)
      return 0
    lax.fori_loop(0, npair, _proc, jnp.int32(0))
    pltpu.make_async_copy(x_hbm.at[own_src[pl.ds(0, L)]], xrows, sem).wait()

    pltpu.sync_copy(agg_own, agg_hbm.at[pl.ds(lo * HEADS + t * AGR, AGR), :])
    plsc.subcore_barrier()
    return 0

  lax.fori_loop(0, CPS, _pass, 0)


_sc_edge = pl.kernel(
    _sc_body,
    out_type=jax.ShapeDtypeStruct((NCHUNK * CN * HEADS, IN), jnp.float32),
    mesh=plsc.VectorSubcoreMesh(core_axis_name="c", subcore_axis_name="s",
                                num_cores=NC, num_subcores=NS),
    compiler_params=pltpu.CompilerParams(needs_layout_passes=False),
    scratch_types=[
        pltpu.VMEM_SHARED((NS * WCAP,), jnp.int32),
        pltpu.VMEM_SHARED((NS * WCAP,), jnp.int32),
        pltpu.VMEM_SHARED((NS * 128,), jnp.int32),
        pltpu.VMEM((SCAN,), jnp.int32),
        pltpu.VMEM((SCAN,), jnp.int32),
        pltpu.VMEM((WCAP,), jnp.int32),
        pltpu.VMEM((WCAP,), jnp.int32),
        pltpu.VMEM((128,), jnp.int32),
        pltpu.VMEM((NS * 128,), jnp.int32),
        pltpu.VMEM((NS * WCAP,), jnp.int32),
        pltpu.VMEM((NS * WCAP,), jnp.int32),
        pltpu.VMEM((OCAP + B1,), jnp.int32),
        pltpu.VMEM((OCAP + B1,), jnp.int32),
        pltpu.VMEM((HEADS * OCAP + B1,), jnp.float32),
        pltpu.VMEM((NPC,), jnp.int32),
        pltpu.VMEM((NPC, 128), jnp.float32),
        pltpu.VMEM((B1, 128), jnp.float32),
        pltpu.VMEM((B1,), jnp.int32),
        pltpu.VMEM((NPC * HEADS * L,), jnp.float32),
        pltpu.VMEM((NPC * HEADS, L), jnp.float32),
        pltpu.VMEM((AGR, IN), jnp.float32),
        pltpu.VMEM((L, IN), jnp.float32),
        pltpu.VMEM((L, IN), jnp.float32),
        pltpu.VMEM((HEADS, L), jnp.float32),
        pltpu.SemaphoreType.DMA,
        pltpu.SemaphoreType.DMA,
    ],
)


def _alpha_vecs(W_ref, as_ref, ad_ref):
  vs, vd = [], []
  for h in range(HEADS):
    Wh = W_ref[:, h * HID:(h + 1) * HID]
    sb = jnp.broadcast_to(as_ref[h:h + 1, :], (IN, HID))
    db = jnp.broadcast_to(ad_ref[h:h + 1, :], (IN, HID))
    vs.append(jnp.sum(Wh * sb, axis=1, keepdims=True))
    vd.append(jnp.sum(Wh * db, axis=1, keepdims=True))
  return (jnp.concatenate(vs, axis=1), jnp.concatenate(vd, axis=1))


def _sad_body(x_ref, W_ref, as_ref, ad_ref, o_ref):
  Vs, Vd = _alpha_vecs(W_ref, as_ref, ad_ref)
  xb = x_ref[...]
  a_s = jnp.dot(xb, Vs, preferred_element_type=jnp.float32)
  a_d = jnp.dot(xb, Vd, preferred_element_type=jnp.float32)
  pad = jnp.zeros((xb.shape[0], 120), jnp.float32)
  o_ref[...] = jnp.concatenate([a_s, a_d, pad], axis=1)


def _sad_call(x, W, att_s, att_d):
  return pl.pallas_call(
      _sad_body,
      grid=(10,),
      in_specs=[
          pl.BlockSpec((1000, IN), lambda i: (i, 0)),
          pl.BlockSpec((IN, HEADS * HID), lambda i: (0, 0)),
          pl.BlockSpec((HEADS, HID), lambda i: (0, 0)),
          pl.BlockSpec((HEADS, HID), lambda i: (0, 0)),
      ],
      out_specs=pl.BlockSpec((1000, 128), lambda i: (i, 0)),
      out_shape=jax.ShapeDtypeStruct((N, 128), jnp.float32),
  )(x, W, att_s, att_d)


def _dense_body(A_ref, Ws_ref, b_ref, g_ref, be_ref, W2_ref, as_ref, ad_ref,
                h_ref, sad_ref):
  acc = jnp.dot(A_ref[...], Ws_ref[...], preferred_element_type=jnp.float32)
  y = (acc * 0.25 + b_ref[...]) * (g_ref[...] * _BN_C) + be_ref[...]
  hv = jnp.maximum(y, 0.0)
  h_ref[...] = hv
  Vs, Vd = _alpha_vecs(W2_ref, as_ref, ad_ref)
  a_s = jnp.dot(hv, Vs, preferred_element_type=jnp.float32)
  a_d = jnp.dot(hv, Vd, preferred_element_type=jnp.float32)
  pad = jnp.zeros((hv.shape[0], 120), jnp.float32)
  sad_ref[...] = jnp.concatenate([a_s, a_d, pad], axis=1)


def _dense_call(A, Ws, b, g, be, W2, att_s2, att_d2):
  return pl.pallas_call(
      _dense_body,
      grid=(10,),
      in_specs=[
          pl.BlockSpec((1000, HEADS * IN), lambda i: (i, 0)),
          pl.BlockSpec((HEADS * IN, HID), lambda i: (0, 0)),
          pl.BlockSpec((1, HID), lambda i: (0, 0)),
          pl.BlockSpec((1, HID), lambda i: (0, 0)),
          pl.BlockSpec((1, HID), lambda i: (0, 0)),
          pl.BlockSpec((HID, HEADS * HID), lambda i: (0, 0)),
          pl.BlockSpec((HEADS, HID), lambda i: (0, 0)),
          pl.BlockSpec((HEADS, HID), lambda i: (0, 0)),
      ],
      out_specs=[
          pl.BlockSpec((1000, HID), lambda i: (i, 0)),
          pl.BlockSpec((1000, 128), lambda i: (i, 0)),
      ],
      out_shape=[
          jax.ShapeDtypeStruct((N, HID), jnp.float32),
          jax.ShapeDtypeStruct((N, 128), jnp.float32),
      ],
  )(A, Ws, b, g, be, W2, att_s2, att_d2)


def _final_body(A_ref, Ws_ref, b_ref, g_ref, be_ref, bat_ref, fw_ref, fb_ref,
                o_ref, sums, counts):
  i = pl.program_id(0)
  acc = jnp.dot(A_ref[...], Ws_ref[...], preferred_element_type=jnp.float32)
  y = (acc * 0.25 + b_ref[...]) * (g_ref[...] * _BN_C) + be_ref[...]
  hv = jnp.maximum(y, 0.0)

  bvec = bat_ref[0, 0, :]
  gid = lax.broadcasted_iota(jnp.int32, (G, 1000), 0)
  onehot = jnp.where(gid == bvec[None, :], 1.0, 0.0).astype(jnp.float32)

  @pl.when(i == 0)
  def _init():
    sums[...] = jnp.zeros_like(sums)
    counts[...] = jnp.zeros_like(counts)

  sums[...] += jnp.dot(onehot, hv, preferred_element_type=jnp.float32)
  counts[...] += jnp.broadcast_to(
      jnp.sum(onehot, axis=1, keepdims=True), (G, HID))

  @pl.when(i == 9)
  def _fin():
    pooled = sums[...] / jnp.maximum(counts[...], 1.0)
    o_ref[...] = (jnp.dot(pooled, fw_ref[...], preferred_element_type=jnp.float32)
                  + fb_ref[...])


def _final_call(A, Ws, b, g, be, batch3, fw, fb):
  return pl.pallas_call(
      _final_body,
      grid=(10,),
      in_specs=[
          pl.BlockSpec((1000, HEADS * HID), lambda i: (i, 0)),
          pl.BlockSpec((HEADS * HID, HID), lambda i: (0, 0)),
          pl.BlockSpec((1, HID), lambda i: (0, 0)),
          pl.BlockSpec((1, HID), lambda i: (0, 0)),
          pl.BlockSpec((1, HID), lambda i: (0, 0)),
          pl.BlockSpec((1, 1, 1000), lambda i: (i, 0, 0)),
          pl.BlockSpec((HID, OUT), lambda i: (0, 0)),
          pl.BlockSpec((1, OUT), lambda i: (0, 0)),
      ],
      out_specs=pl.BlockSpec((G, OUT), lambda i: (0, 0)),
      out_shape=jax.ShapeDtypeStruct((G, OUT), jnp.float32),
      scratch_shapes=[
          pltpu.VMEM((G, HID), jnp.float32),
          pltpu.VMEM((G, HID), jnp.float32),
      ],
  )(A, Ws, b, g, be, batch3, fw, fb)


def kernel(x, edge_index, batch, W1, att_src1, att_dst1, bias1,
           W2, att_src2, att_dst2, bias2,
           bn1_gamma, bn1_beta, bn2_gamma, bn2_beta, fc_W, fc_b):
  src = edge_index[0].astype(jnp.int32)
  dst = edge_index[1].astype(jnp.int32)
  batch3 = batch.astype(jnp.int32).reshape(10, 1, 1000)

  Ws1 = W1.reshape(IN, HEADS, HID).transpose(1, 0, 2).reshape(HEADS * IN, HID)
  Ws2 = W2.reshape(HID, HEADS, HID).transpose(1, 0, 2).reshape(HEADS * HID, HID)
  b1 = bias1.reshape(1, HID)
  b2 = bias2.reshape(1, HID)
  g1 = bn1_gamma.reshape(1, HID)
  g2 = bn2_gamma.reshape(1, HID)
  be1 = bn1_beta.reshape(1, HID)
  be2 = bn2_beta.reshape(1, HID)
  fbr = fc_b.reshape(1, OUT)

  sad1 = _sad_call(x, W1, att_src1, att_dst1)
  agg1 = _sc_edge(sad1, src, dst, x)
  h1, sad2 = _dense_call(agg1.reshape(-1, HEADS * IN), Ws1, b1, g1, be1,
                         W2, att_src2, att_dst2)
  agg2 = _sc_edge(sad2, src, dst, h1)
  return _final_call(agg2.reshape(-1, HEADS * HID), Ws2, b2, g2, be2,
                     batch3, fc_W, fbr)


# speedup vs baseline: 14.8618x; 1.0456x over previous
import functools

import jax
import jax.numpy as jnp
from jax import lax
from jax.experimental import pallas as pl
from jax.experimental.pallas import tpu as pltpu
from jax.experimental.pallas import tpu_sc as plsc

N = 10000
E = 160000
IN = 256
HID = 256
HEADS = 4
OUT = 128
G = 64

NC = 2
NS = 16
L = 16

EPT = E // NS
SUB = 400
NSUB = EPT // SUB
SCAN = 2000
NSCAN = EPT // SCAN
NCHUNK = 20
CPS = NCHUNK // NC
CN = 512
CROWS = CN * HEADS
RPT = CROWS // NS
WCAP = 1024
DENR = 64
NPT = N // NS
KV = IN // L

import math
_BN_C = 1.0 / math.sqrt(1.0 + 1e-5)


B1 = 64
WCAP = 1024
OCAP = 1024
NPC = 32
AGR = NPC * HEADS


def _sc_body(sad_hbm, src_hbm, dst_hbm, x_hbm, agg_hbm,
             wlsp_src, wlsp_dst, cnts_sp,
             sblk, dblk, wl_src, wl_dstl, cntob, cntib,
             in_src, in_dst, own_src, own_dstl, own_ex,
             idx32, adst_own, gb_s, s64,
             den_own, den_red, agg_own, xrows, xrows2, cbuf, sem, sem2):
  c = lax.axis_index("c")
  t = lax.axis_index("s")
  ebase = t * EPT
  iot = lax.iota(jnp.int32, L)
  zf = jnp.zeros((L,), jnp.float32)
  zi = jnp.zeros((L,), jnp.int32)

  pltpu.sync_copy(src_hbm.at[pl.ds(ebase, EPT)], sblk)
  pltpu.sync_copy(dst_hbm.at[pl.ds(ebase, EPT)], dblk)

  def _pass(p, _):
    chunk = c * CPS + p
    lo = chunk * CN
    obase = lo + t * NPC

    def _zagg(r, _):
      for k in range(KV):
        agg_own[r, pl.ds(k * L, L)] = zf
      return 0
    lax.fori_loop(0, AGR, _zagg, 0)

    def _zden(r, _):
      den_own[pl.ds(r * L, L)] = zf
      return 0
    lax.fori_loop(0, NPC * HEADS * L // L, _zden, 0)

    def _cgrp(g, cnt):
      o = g * L
      dv = dblk[pl.ds(o, L)]
      m = (dv >= lo) & (dv < lo + CN)
      cs = plsc.cumsum(jnp.where(m, 1, 0))
      pos = cs + (cnt - 1)
      plsc.store_scatter(wl_src, [pos], sblk[pl.ds(o, L)], mask=m)
      plsc.store_scatter(wl_dstl, [pos], dv - lo, mask=m)
      return cnt + jnp.max(cs)
    cnt = lax.fori_loop(0, EPT // L, _cgrp, jnp.int32(0))

    cntob[pl.ds(0, L)] = jnp.full((L,), 0, jnp.int32) + cnt
    pltpu.sync_copy(wl_src, wlsp_src.at[pl.ds(t * WCAP, WCAP)])
    pltpu.sync_copy(wl_dstl, wlsp_dst.at[pl.ds(t * WCAP, WCAP)])
    pltpu.sync_copy(cntob, cnts_sp.at[pl.ds(t * 128, 128)])
    plsc.subcore_barrier()

    pltpu.sync_copy(cnts_sp, cntib)

    def _pullh(hh, ocnt):
      pltpu.sync_copy(wlsp_src.at[pl.ds(hh * 8 * WCAP, 8 * WCAP)], in_src)
      pltpu.sync_copy(wlsp_dst.at[pl.ds(hh * 8 * WCAP, 8 * WCAP)], in_dst)

      def _pull(t2, ocnt):
        cv = cntib[pl.ds((hh * 8 + t2) * 128, L)]
        cnt2 = cv[0]
        ng2 = (cnt2 + (L - 1)) // L
        tb = t2 * WCAP

        def _fgrp(g, ocnt):
          o = tb + g * L
          dv = in_dst[pl.ds(o, L)]
          m = ((dv >> 5) == t) & ((iot + g * L) < cnt2)
          cs = plsc.cumsum(jnp.where(m, 1, 0))
          pos = cs + (ocnt - 1)
          plsc.store_scatter(own_src, [pos], in_src[pl.ds(o, L)], mask=m)
          plsc.store_scatter(own_dstl, [pos], dv & (NPC - 1), mask=m)
          return ocnt + jnp.max(cs)
        return lax.fori_loop(0, ng2, _fgrp, ocnt)
      return lax.fori_loop(0, NS // 2, _pull, ocnt)
    ocnt = lax.fori_loop(0, 2, _pullh, jnp.int32(0))

    def _pad(k, _):
      own_src[pl.ds(ocnt + k * L, L)] = zi
      own_dstl[pl.ds(ocnt + k * L, L)] = zi
      return 0
    lax.fori_loop(0, B1 // L, _pad, 0)

    idx32[pl.ds(0, L)] = jnp.minimum(iot + obase, N - 1)
    idx32[pl.ds(L, L)] = jnp.minimum(iot + (obase + L), N - 1)
    pltpu.async_copy(sad_hbm.at[idx32], adst_own, sem).wait()

    nb1 = (ocnt + (B1 - 1)) // B1

    def _w1(b, _):
      o = b * B1

      def _prep(k, _):
        s64[pl.ds(k * L, L)] = own_src[pl.ds(o + k * L, L)]
        return 0
      lax.fori_loop(0, B1 // L, _prep, 0)
      pltpu.async_copy(sad_hbm.at[s64], gb_s, sem).wait()

      def _grp(g, _):
        rid = iot + g * L
        validm = (rid + o) < ocnt
        dl = own_dstl[pl.ds(o + g * L, L)]
        for h in range(HEADS):
          fh = jnp.full((L,), h, jnp.int32)
          asr = plsc.load_gather(gb_s, [rid, fh])
          ads = plsc.load_gather(adst_own, [dl, jnp.full((L,), h + HEADS, jnp.int32)])
          a = asr + ads
          a = jnp.maximum(a, a * 0.2)
          exv = jnp.where(validm, jnp.exp(a), 0.0)
          own_ex[pl.ds(h * OCAP + o + g * L, L)] = exv
          plsc.addupdate_scatter(den_own, [dl * (HEADS * L) + h * L + iot], exv)
        return 0
      lax.fori_loop(0, B1 // L, _grp, 0)
      return 0
    lax.fori_loop(0, nb1, _w1, 0)

    def _dred(r, _):
      dsum = jnp.sum(den_own[pl.ds(r * L, L)])
      den_red[r, :] = jnp.full((L,), 0.0, jnp.float32) + dsum
      return 0
    lax.fori_loop(0, NPC * HEADS, _dred, 0)

    npair = (ocnt + (2 * L - 1)) // (2 * L)

    def _half(o, xr):
      dl = own_dstl[pl.ds(o, L)]
      zc = jnp.full((L,), 0, jnp.int32)
      for h in range(HEADS):
        exv = own_ex[pl.ds(h * OCAP + o, L)]
        dnv = plsc.load_gather(den_red, [dl * HEADS + h, zc])
        cbuf[h, :] = exv / jnp.maximum(dnv, 1e-30)

      def _pere(e, _):
        row = [xr[e, pl.ds(k * L, L)] for k in range(KV)]
        ev = jnp.full((L,), 0, jnp.int32) + e
        dle = plsc.load_gather(own_dstl, [ev + o])
        for h in range(HEADS):
          ch = plsc.load_gather(cbuf, [jnp.full((L,), h, jnp.int32), ev])
          r4 = dle * HEADS + h
          for k in range(KV):
            plsc.addupdate_scatter(agg_own, [r4, iot + k * L], row[k] * ch)
        return 0
      lax.fori_loop(0, L, _pere, 0)

    cp0 = pltpu.async_copy(x_hbm.at[own_src[pl.ds(0, L)]], xrows, sem)

    def _proc(j, _):
      o = 2 * j * L
      cpb = pltpu.async_copy(x_hbm.at[own_src[pl.ds(o + L, L)]], xrows2, sem2)
      pltpu.make_async_copy(x_hbm.at[own_src[pl.ds(o, L)]], xrows, sem).wait()
      _half(o, xrows)
      cpa = pltpu.async_copy(x_hbm.at[own_src[pl.ds(o + 2 * L, L)]], xrows, sem)
      pltpu.make_async_copy(x_hbm.at[own_src[pl.ds(o + L, L)]], xrows2, sem2).wait()
      _half(o + L, xrows2)
      return 0
    lax.fori_loop(0, npair, _proc, jnp.int32(0))
    pltpu.make_async_copy(x_hbm.at[own_src[pl.ds(0, L)]], xrows, sem).wait()

    pltpu.sync_copy(agg_own, agg_hbm.at[pl.ds(lo * HEADS + t * AGR, AGR), :])
    plsc.subcore_barrier()
    return 0

  lax.fori_loop(0, CPS, _pass, 0)


_sc_edge = pl.kernel(
    _sc_body,
    out_type=jax.ShapeDtypeStruct((NCHUNK * CN * HEADS, IN), jnp.float32),
    mesh=plsc.VectorSubcoreMesh(core_axis_name="c", subcore_axis_name="s",
                                num_cores=NC, num_subcores=NS),
    compiler_params=pltpu.CompilerParams(needs_layout_passes=False),
    scratch_types=[
        pltpu.VMEM_SHARED((NS * WCAP,), jnp.int32),
        pltpu.VMEM_SHARED((NS * WCAP,), jnp.int32),
        pltpu.VMEM_SHARED((NS * 128,), jnp.int32),
        pltpu.VMEM((EPT,), jnp.int32),
        pltpu.VMEM((EPT,), jnp.int32),
        pltpu.VMEM((WCAP,), jnp.int32),
        pltpu.VMEM((WCAP,), jnp.int32),
        pltpu.VMEM((128,), jnp.int32),
        pltpu.VMEM((NS * 128,), jnp.int32),
        pltpu.VMEM((8 * WCAP,), jnp.int32),
        pltpu.VMEM((8 * WCAP,), jnp.int32),
        pltpu.VMEM((OCAP + B1,), jnp.int32),
        pltpu.VMEM((OCAP + B1,), jnp.int32),
        pltpu.VMEM((HEADS * OCAP + B1,), jnp.float32),
        pltpu.VMEM((NPC,), jnp.int32),
        pltpu.VMEM((NPC, 128), jnp.float32),
        pltpu.VMEM((B1, 128), jnp.float32),
        pltpu.VMEM((B1,), jnp.int32),
        pltpu.VMEM((NPC * HEADS * L,), jnp.float32),
        pltpu.VMEM((NPC * HEADS, L), jnp.float32),
        pltpu.VMEM((AGR, IN), jnp.float32),
        pltpu.VMEM((L, IN), jnp.float32),
        pltpu.VMEM((L, IN), jnp.float32),
        pltpu.VMEM((HEADS, L), jnp.float32),
        pltpu.SemaphoreType.DMA,
        pltpu.SemaphoreType.DMA,
    ],
)


def _alpha_vecs(W_ref, as_ref, ad_ref):
  vs, vd = [], []
  for h in range(HEADS):
    Wh = W_ref[:, h * HID:(h + 1) * HID]
    sb = jnp.broadcast_to(as_ref[h:h + 1, :], (IN, HID))
    db = jnp.broadcast_to(ad_ref[h:h + 1, :], (IN, HID))
    vs.append(jnp.sum(Wh * sb, axis=1, keepdims=True))
    vd.append(jnp.sum(Wh * db, axis=1, keepdims=True))
  return (jnp.concatenate(vs, axis=1), jnp.concatenate(vd, axis=1))


def _sad_body(x_ref, W_ref, as_ref, ad_ref, o_ref):
  Vs, Vd = _alpha_vecs(W_ref, as_ref, ad_ref)
  xb = x_ref[...]
  a_s = jnp.dot(xb, Vs, preferred_element_type=jnp.float32)
  a_d = jnp.dot(xb, Vd, preferred_element_type=jnp.float32)
  pad = jnp.zeros((xb.shape[0], 120), jnp.float32)
  o_ref[...] = jnp.concatenate([a_s, a_d, pad], axis=1)


def _sad_call(x, W, att_s, att_d):
  return pl.pallas_call(
      _sad_body,
      grid=(10,),
      in_specs=[
          pl.BlockSpec((1000, IN), lambda i: (i, 0)),
          pl.BlockSpec((IN, HEADS * HID), lambda i: (0, 0)),
          pl.BlockSpec((HEADS, HID), lambda i: (0, 0)),
          pl.BlockSpec((HEADS, HID), lambda i: (0, 0)),
      ],
      out_specs=pl.BlockSpec((1000, 128), lambda i: (i, 0)),
      out_shape=jax.ShapeDtypeStruct((N, 128), jnp.float32),
  )(x, W, att_s, att_d)


def _dense_body(A_ref, Ws_ref, b_ref, g_ref, be_ref, W2_ref, as_ref, ad_ref,
                h_ref, sad_ref):
  acc = jnp.dot(A_ref[...], Ws_ref[...], preferred_element_type=jnp.float32)
  y = (acc * 0.25 + b_ref[...]) * (g_ref[...] * _BN_C) + be_ref[...]
  hv = jnp.maximum(y, 0.0)
  h_ref[...] = hv
  Vs, Vd = _alpha_vecs(W2_ref, as_ref, ad_ref)
  a_s = jnp.dot(hv, Vs, preferred_element_type=jnp.float32)
  a_d = jnp.dot(hv, Vd, preferred_element_type=jnp.float32)
  pad = jnp.zeros((hv.shape[0], 120), jnp.float32)
  sad_ref[...] = jnp.concatenate([a_s, a_d, pad], axis=1)


def _dense_call(A, Ws, b, g, be, W2, att_s2, att_d2):
  return pl.pallas_call(
      _dense_body,
      grid=(10,),
      in_specs=[
          pl.BlockSpec((1000, HEADS * IN), lambda i: (i, 0)),
          pl.BlockSpec((HEADS * IN, HID), lambda i: (0, 0)),
          pl.BlockSpec((1, HID), lambda i: (0, 0)),
          pl.BlockSpec((1, HID), lambda i: (0, 0)),
          pl.BlockSpec((1, HID), lambda i: (0, 0)),
          pl.BlockSpec((HID, HEADS * HID), lambda i: (0, 0)),
          pl.BlockSpec((HEADS, HID), lambda i: (0, 0)),
          pl.BlockSpec((HEADS, HID), lambda i: (0, 0)),
      ],
      out_specs=[
          pl.BlockSpec((1000, HID), lambda i: (i, 0)),
          pl.BlockSpec((1000, 128), lambda i: (i, 0)),
      ],
      out_shape=[
          jax.ShapeDtypeStruct((N, HID), jnp.float32),
          jax.ShapeDtypeStruct((N, 128), jnp.float32),
      ],
  )(A, Ws, b, g, be, W2, att_s2, att_d2)


def _final_body(A_ref, Ws_ref, b_ref, g_ref, be_ref, bat_ref, fw_ref, fb_ref,
                o_ref, sums, counts):
  i = pl.program_id(0)
  acc = jnp.dot(A_ref[...], Ws_ref[...], preferred_element_type=jnp.float32)
  y = (acc * 0.25 + b_ref[...]) * (g_ref[...] * _BN_C) + be_ref[...]
  hv = jnp.maximum(y, 0.0)

  bvec = bat_ref[0, 0, :]
  gid = lax.broadcasted_iota(jnp.int32, (G, 1000), 0)
  onehot = jnp.where(gid == bvec[None, :], 1.0, 0.0).astype(jnp.float32)

  @pl.when(i == 0)
  def _init():
    sums[...] = jnp.zeros_like(sums)
    counts[...] = jnp.zeros_like(counts)

  sums[...] += jnp.dot(onehot, hv, preferred_element_type=jnp.float32)
  counts[...] += jnp.broadcast_to(
      jnp.sum(onehot, axis=1, keepdims=True), (G, HID))

  @pl.when(i == 9)
  def _fin():
    pooled = sums[...] / jnp.maximum(counts[...], 1.0)
    o_ref[...] = (jnp.dot(pooled, fw_ref[...], preferred_element_type=jnp.float32)
                  + fb_ref[...])


def _final_call(A, Ws, b, g, be, batch3, fw, fb):
  return pl.pallas_call(
      _final_body,
      grid=(10,),
      in_specs=[
          pl.BlockSpec((1000, HEADS * HID), lambda i: (i, 0)),
          pl.BlockSpec((HEADS * HID, HID), lambda i: (0, 0)),
          pl.BlockSpec((1, HID), lambda i: (0, 0)),
          pl.BlockSpec((1, HID), lambda i: (0, 0)),
          pl.BlockSpec((1, HID), lambda i: (0, 0)),
          pl.BlockSpec((1, 1, 1000), lambda i: (i, 0, 0)),
          pl.BlockSpec((HID, OUT), lambda i: (0, 0)),
          pl.BlockSpec((1, OUT), lambda i: (0, 0)),
      ],
      out_specs=pl.BlockSpec((G, OUT), lambda i: (0, 0)),
      out_shape=jax.ShapeDtypeStruct((G, OUT), jnp.float32),
      scratch_shapes=[
          pltpu.VMEM((G, HID), jnp.float32),
          pltpu.VMEM((G, HID), jnp.float32),
      ],
  )(A, Ws, b, g, be, batch3, fw, fb)


def kernel(x, edge_index, batch, W1, att_src1, att_dst1, bias1,
           W2, att_src2, att_dst2, bias2,
           bn1_gamma, bn1_beta, bn2_gamma, bn2_beta, fc_W, fc_b):
  src = edge_index[0].astype(jnp.int32)
  dst = edge_index[1].astype(jnp.int32)
  batch3 = batch.astype(jnp.int32).reshape(10, 1, 1000)

  Ws1 = W1.reshape(IN, HEADS, HID).transpose(1, 0, 2).reshape(HEADS * IN, HID)
  Ws2 = W2.reshape(HID, HEADS, HID).transpose(1, 0, 2).reshape(HEADS * HID, HID)
  b1 = bias1.reshape(1, HID)
  b2 = bias2.reshape(1, HID)
  g1 = bn1_gamma.reshape(1, HID)
  g2 = bn2_gamma.reshape(1, HID)
  be1 = bn1_beta.reshape(1, HID)
  be2 = bn2_beta.reshape(1, HID)
  fbr = fc_b.reshape(1, OUT)

  sad1 = _sad_call(x, W1, att_src1, att_dst1)
  agg1 = _sc_edge(sad1, src, dst, x)
  h1, sad2 = _dense_call(agg1.reshape(-1, HEADS * IN), Ws1, b1, g1, be1,
                         W2, att_src2, att_dst2)
  agg2 = _sc_edge(sad2, src, dst, h1)
  return _final_call(agg2.reshape(-1, HEADS * HID), Ws2, b2, g2, be2,
                     batch3, fc_W, fbr)


# speedup vs baseline: 14.9961x; 1.0090x over previous
import functools

import jax
import jax.numpy as jnp
from jax import lax
from jax.experimental import pallas as pl
from jax.experimental.pallas import tpu as pltpu
from jax.experimental.pallas import tpu_sc as plsc

N = 10000
E = 160000
IN = 256
HID = 256
HEADS = 4
OUT = 128
G = 64

NC = 2
NS = 16
L = 16

EPT = E // NS
SUB = 400
NSUB = EPT // SUB
SCAN = 2000
NSCAN = EPT // SCAN
NCHUNK = 20
CPS = NCHUNK // NC
CN = 512
CROWS = CN * HEADS
RPT = CROWS // NS
WCAP = 1024
DENR = 64
NPT = N // NS
KV = IN // L

import math
_BN_C = 1.0 / math.sqrt(1.0 + 1e-5)


B1 = 64
WCAP = 1024
OCAP = 1024
NPC = 32
AGR = NPC * HEADS


def _sc_body(sad_hbm, src_hbm, dst_hbm, x_hbm, agg_hbm,
             wlsp_src, wlsp_dst, cnts_sp,
             sblk, dblk, wl_src, wl_dstl, cntob, cntib,
             in_src, in_dst, own_src, own_dstl, own_ex,
             idx32, adst_own, gb_s, s64,
             den_own, den_red, agg_own, xrows, xrows2, cbuf, sem, sem2):
  c = lax.axis_index("c")
  t = lax.axis_index("s")
  ebase = t * EPT
  iot = lax.iota(jnp.int32, L)
  zf = jnp.zeros((L,), jnp.float32)
  zi = jnp.zeros((L,), jnp.int32)

  pltpu.sync_copy(src_hbm.at[pl.ds(ebase, EPT)], sblk)
  pltpu.sync_copy(dst_hbm.at[pl.ds(ebase, EPT)], dblk)

  def _pass(p, _):
    chunk = c * CPS + p
    lo = chunk * CN
    obase = lo + t * NPC

    def _zagg(r, _):
      for k in range(KV):
        agg_own[r, pl.ds(k * L, L)] = zf
      return 0
    lax.fori_loop(0, AGR, _zagg, 0)

    def _zden(r, _):
      den_own[pl.ds(r * L, L)] = zf
      return 0
    lax.fori_loop(0, NPC * HEADS * L // L, _zden, 0)

    def _cgrp(g, cnt):
      o = g * L
      dv = dblk[pl.ds(o, L)]
      m = (dv >= lo) & (dv < lo + CN)
      cs = plsc.cumsum(jnp.where(m, 1, 0))
      pos = cs + (cnt - 1)
      plsc.store_scatter(wl_src, [pos], sblk[pl.ds(o, L)], mask=m)
      plsc.store_scatter(wl_dstl, [pos], dv - lo, mask=m)
      return cnt + cs[L - 1]
    cnt = lax.fori_loop(0, EPT // L, _cgrp, jnp.int32(0))

    cntob[pl.ds(0, L)] = jnp.full((L,), 0, jnp.int32) + cnt
    pltpu.sync_copy(wl_src, wlsp_src.at[pl.ds(t * WCAP, WCAP)])
    pltpu.sync_copy(wl_dstl, wlsp_dst.at[pl.ds(t * WCAP, WCAP)])
    pltpu.sync_copy(cntob, cnts_sp.at[pl.ds(t * 128, 128)])
    plsc.subcore_barrier()

    pltpu.sync_copy(cnts_sp, cntib)

    def _pullh(hh, ocnt):
      pltpu.sync_copy(wlsp_src.at[pl.ds(hh * 8 * WCAP, 8 * WCAP)], in_src)
      pltpu.sync_copy(wlsp_dst.at[pl.ds(hh * 8 * WCAP, 8 * WCAP)], in_dst)

      def _pull(t2, ocnt):
        cv = cntib[pl.ds((hh * 8 + t2) * 128, L)]
        cnt2 = cv[0]
        ng2 = (cnt2 + (L - 1)) // L
        tb = t2 * WCAP

        def _fgrp(g, ocnt):
          o = tb + g * L
          dv = in_dst[pl.ds(o, L)]
          m = ((dv >> 5) == t) & ((iot + g * L) < cnt2)
          cs = plsc.cumsum(jnp.where(m, 1, 0))
          pos = cs + (ocnt - 1)
          plsc.store_scatter(own_src, [pos], in_src[pl.ds(o, L)], mask=m)
          plsc.store_scatter(own_dstl, [pos], dv & (NPC - 1), mask=m)
          return ocnt + cs[L - 1]
        return lax.fori_loop(0, ng2, _fgrp, ocnt)
      return lax.fori_loop(0, NS // 2, _pull, ocnt)
    ocnt = lax.fori_loop(0, 2, _pullh, jnp.int32(0))

    def _pad(k, _):
      own_src[pl.ds(ocnt + k * L, L)] = zi
      own_dstl[pl.ds(ocnt + k * L, L)] = zi
      return 0
    lax.fori_loop(0, B1 // L, _pad, 0)

    idx32[pl.ds(0, L)] = jnp.minimum(iot + obase, N - 1)
    idx32[pl.ds(L, L)] = jnp.minimum(iot + (obase + L), N - 1)
    pltpu.async_copy(sad_hbm.at[idx32], adst_own, sem).wait()

    nb1 = (ocnt + (B1 - 1)) // B1

    def _w1(b, _):
      o = b * B1

      def _prep(k, _):
        s64[pl.ds(k * L, L)] = own_src[pl.ds(o + k * L, L)]
        return 0
      lax.fori_loop(0, B1 // L, _prep, 0)
      pltpu.async_copy(sad_hbm.at[s64], gb_s, sem).wait()

      def _grp(g, _):
        rid = iot + g * L
        validm = (rid + o) < ocnt
        dl = own_dstl[pl.ds(o + g * L, L)]
        for h in range(HEADS):
          fh = jnp.full((L,), h, jnp.int32)
          asr = plsc.load_gather(gb_s, [rid, fh])
          ads = plsc.load_gather(adst_own, [dl, jnp.full((L,), h + HEADS, jnp.int32)])
          a = asr + ads
          a = jnp.maximum(a, a * 0.2)
          exv = jnp.where(validm, jnp.exp(a), 0.0)
          own_ex[pl.ds(h * OCAP + o + g * L, L)] = exv
          plsc.addupdate_scatter(den_own, [dl * (HEADS * L) + h * L + iot], exv)
        return 0
      lax.fori_loop(0, B1 // L, _grp, 0)
      return 0
    lax.fori_loop(0, nb1, _w1, 0)

    def _dred(r, _):
      dsum = jnp.sum(den_own[pl.ds(r * L, L)])
      den_red[r, :] = jnp.full((L,), 0.0, jnp.float32) + dsum
      return 0
    lax.fori_loop(0, NPC * HEADS, _dred, 0)

    npair = (ocnt + (2 * L - 1)) // (2 * L)

    def _half(o, xr):
      dl = own_dstl[pl.ds(o, L)]
      zc = jnp.full((L,), 0, jnp.int32)
      for h in range(HEADS):
        exv = own_ex[pl.ds(h * OCAP + o, L)]
        dnv = plsc.load_gather(den_red, [dl * HEADS + h, zc])
        cbuf[h, :] = exv / jnp.maximum(dnv, 1e-30)

      def _pere(e, _):
        row = [xr[e, pl.ds(k * L, L)] for k in range(KV)]
        ev = jnp.full((L,), 0, jnp.int32) + e
        dle = plsc.load_gather(own_dstl, [ev + o])
        for h in range(HEADS):
          ch = plsc.load_gather(cbuf, [jnp.full((L,), h, jnp.int32), ev])
          r4 = dle * HEADS + h
          for k in range(KV):
            plsc.addupdate_scatter(agg_own, [r4, iot + k * L], row[k] * ch)
        return 0
      lax.fori_loop(0, L, _pere, 0)

    cp0 = pltpu.async_copy(x_hbm.at[own_src[pl.ds(0, L)]], xrows, sem)

    def _proc(j, _):
      o = 2 * j * L
      cpb = pltpu.async_copy(x_hbm.at[own_src[pl.ds(o + L, L)]], xrows2, sem2)
      pltpu.make_async_copy(x_hbm.at[own_src[pl.ds(o, L)]], xrows, sem).wait()
      _half(o, xrows)
      cpa = pltpu.async_copy(x_hbm.at[own_src[pl.ds(o + 2 * L, L)]], xrows, sem)
      pltpu.make_async_copy(x_hbm.at[own_src[pl.ds(o + L, L)]], xrows2, sem2).wait()
      _half(o + L, xrows2)
      return 0
    lax.fori_loop(0, npair, _proc, jnp.int32(0))
    pltpu.make_async_copy(x_hbm.at[own_src[pl.ds(0, L)]], xrows, sem).wait()

    pltpu.sync_copy(agg_own, agg_hbm.at[pl.ds(lo * HEADS + t * AGR, AGR), :])
    plsc.subcore_barrier()
    return 0

  lax.fori_loop(0, CPS, _pass, 0)


_sc_edge = pl.kernel(
    _sc_body,
    out_type=jax.ShapeDtypeStruct((NCHUNK * CN * HEADS, IN), jnp.float32),
    mesh=plsc.VectorSubcoreMesh(core_axis_name="c", subcore_axis_name="s",
                                num_cores=NC, num_subcores=NS),
    compiler_params=pltpu.CompilerParams(needs_layout_passes=False),
    scratch_types=[
        pltpu.VMEM_SHARED((NS * WCAP,), jnp.int32),
        pltpu.VMEM_SHARED((NS * WCAP,), jnp.int32),
        pltpu.VMEM_SHARED((NS * 128,), jnp.int32),
        pltpu.VMEM((EPT,), jnp.int32),
        pltpu.VMEM((EPT,), jnp.int32),
        pltpu.VMEM((WCAP,), jnp.int32),
        pltpu.VMEM((WCAP,), jnp.int32),
        pltpu.VMEM((128,), jnp.int32),
        pltpu.VMEM((NS * 128,), jnp.int32),
        pltpu.VMEM((8 * WCAP,), jnp.int32),
        pltpu.VMEM((8 * WCAP,), jnp.int32),
        pltpu.VMEM((OCAP + B1,), jnp.int32),
        pltpu.VMEM((OCAP + B1,), jnp.int32),
        pltpu.VMEM((HEADS * OCAP + B1,), jnp.float32),
        pltpu.VMEM((NPC,), jnp.int32),
        pltpu.VMEM((NPC, 128), jnp.float32),
        pltpu.VMEM((B1, 128), jnp.float32),
        pltpu.VMEM((B1,), jnp.int32),
        pltpu.VMEM((NPC * HEADS * L,), jnp.float32),
        pltpu.VMEM((NPC * HEADS, L), jnp.float32),
        pltpu.VMEM((AGR, IN), jnp.float32),
        pltpu.VMEM((L, IN), jnp.float32),
        pltpu.VMEM((L, IN), jnp.float32),
        pltpu.VMEM((HEADS, L), jnp.float32),
        pltpu.SemaphoreType.DMA,
        pltpu.SemaphoreType.DMA,
    ],
)


def _alpha_vecs(W_ref, as_ref, ad_ref):
  vs, vd = [], []
  for h in range(HEADS):
    Wh = W_ref[:, h * HID:(h + 1) * HID]
    sb = jnp.broadcast_to(as_ref[h:h + 1, :], (IN, HID))
    db = jnp.broadcast_to(ad_ref[h:h + 1, :], (IN, HID))
    vs.append(jnp.sum(Wh * sb, axis=1, keepdims=True))
    vd.append(jnp.sum(Wh * db, axis=1, keepdims=True))
  return (jnp.concatenate(vs, axis=1), jnp.concatenate(vd, axis=1))


def _sad_body(x_ref, W_ref, as_ref, ad_ref, o_ref):
  Vs, Vd = _alpha_vecs(W_ref, as_ref, ad_ref)
  xb = x_ref[...]
  a_s = jnp.dot(xb, Vs, preferred_element_type=jnp.float32)
  a_d = jnp.dot(xb, Vd, preferred_element_type=jnp.float32)
  pad = jnp.zeros((xb.shape[0], 120), jnp.float32)
  o_ref[...] = jnp.concatenate([a_s, a_d, pad], axis=1)


def _sad_call(x, W, att_s, att_d):
  return pl.pallas_call(
      _sad_body,
      grid=(10,),
      in_specs=[
          pl.BlockSpec((1000, IN), lambda i: (i, 0)),
          pl.BlockSpec((IN, HEADS * HID), lambda i: (0, 0)),
          pl.BlockSpec((HEADS, HID), lambda i: (0, 0)),
          pl.BlockSpec((HEADS, HID), lambda i: (0, 0)),
      ],
      out_specs=pl.BlockSpec((1000, 128), lambda i: (i, 0)),
      out_shape=jax.ShapeDtypeStruct((N, 128), jnp.float32),
  )(x, W, att_s, att_d)


def _dense_body(A_ref, Ws_ref, b_ref, g_ref, be_ref, W2_ref, as_ref, ad_ref,
                h_ref, sad_ref):
  acc = jnp.dot(A_ref[...], Ws_ref[...], preferred_element_type=jnp.float32)
  y = (acc * 0.25 + b_ref[...]) * (g_ref[...] * _BN_C) + be_ref[...]
  hv = jnp.maximum(y, 0.0)
  h_ref[...] = hv
  Vs, Vd = _alpha_vecs(W2_ref, as_ref, ad_ref)
  a_s = jnp.dot(hv, Vs, preferred_element_type=jnp.float32)
  a_d = jnp.dot(hv, Vd, preferred_element_type=jnp.float32)
  pad = jnp.zeros((hv.shape[0], 120), jnp.float32)
  sad_ref[...] = jnp.concatenate([a_s, a_d, pad], axis=1)


def _dense_call(A, Ws, b, g, be, W2, att_s2, att_d2):
  return pl.pallas_call(
      _dense_body,
      grid=(10,),
      in_specs=[
          pl.BlockSpec((1000, HEADS * IN), lambda i: (i, 0)),
          pl.BlockSpec((HEADS * IN, HID), lambda i: (0, 0)),
          pl.BlockSpec((1, HID), lambda i: (0, 0)),
          pl.BlockSpec((1, HID), lambda i: (0, 0)),
          pl.BlockSpec((1, HID), lambda i: (0, 0)),
          pl.BlockSpec((HID, HEADS * HID), lambda i: (0, 0)),
          pl.BlockSpec((HEADS, HID), lambda i: (0, 0)),
          pl.BlockSpec((HEADS, HID), lambda i: (0, 0)),
      ],
      out_specs=[
          pl.BlockSpec((1000, HID), lambda i: (i, 0)),
          pl.BlockSpec((1000, 128), lambda i: (i, 0)),
      ],
      out_shape=[
          jax.ShapeDtypeStruct((N, HID), jnp.float32),
          jax.ShapeDtypeStruct((N, 128), jnp.float32),
      ],
  )(A, Ws, b, g, be, W2, att_s2, att_d2)


def _final_body(A_ref, Ws_ref, b_ref, g_ref, be_ref, bat_ref, fw_ref, fb_ref,
                o_ref, sums, counts):
  i = pl.program_id(0)
  acc = jnp.dot(A_ref[...], Ws_ref[...], preferred_element_type=jnp.float32)
  y = (acc * 0.25 + b_ref[...]) * (g_ref[...] * _BN_C) + be_ref[...]
  hv = jnp.maximum(y, 0.0)

  bvec = bat_ref[0, 0, :]
  gid = lax.broadcasted_iota(jnp.int32, (G, 1000), 0)
  onehot = jnp.where(gid == bvec[None, :], 1.0, 0.0).astype(jnp.float32)

  @pl.when(i == 0)
  def _init():
    sums[...] = jnp.zeros_like(sums)
    counts[...] = jnp.zeros_like(counts)

  sums[...] += jnp.dot(onehot, hv, preferred_element_type=jnp.float32)
  counts[...] += jnp.broadcast_to(
      jnp.sum(onehot, axis=1, keepdims=True), (G, HID))

  @pl.when(i == 9)
  def _fin():
    pooled = sums[...] / jnp.maximum(counts[...], 1.0)
    o_ref[...] = (jnp.dot(pooled, fw_ref[...], preferred_element_type=jnp.float32)
                  + fb_ref[...])


def _final_call(A, Ws, b, g, be, batch3, fw, fb):
  return pl.pallas_call(
      _final_body,
      grid=(10,),
      in_specs=[
          pl.BlockSpec((1000, HEADS * HID), lambda i: (i, 0)),
          pl.BlockSpec((HEADS * HID, HID), lambda i: (0, 0)),
          pl.BlockSpec((1, HID), lambda i: (0, 0)),
          pl.BlockSpec((1, HID), lambda i: (0, 0)),
          pl.BlockSpec((1, HID), lambda i: (0, 0)),
          pl.BlockSpec((1, 1, 1000), lambda i: (i, 0, 0)),
          pl.BlockSpec((HID, OUT), lambda i: (0, 0)),
          pl.BlockSpec((1, OUT), lambda i: (0, 0)),
      ],
      out_specs=pl.BlockSpec((G, OUT), lambda i: (0, 0)),
      out_shape=jax.ShapeDtypeStruct((G, OUT), jnp.float32),
      scratch_shapes=[
          pltpu.VMEM((G, HID), jnp.float32),
          pltpu.VMEM((G, HID), jnp.float32),
      ],
  )(A, Ws, b, g, be, batch3, fw, fb)


def kernel(x, edge_index, batch, W1, att_src1, att_dst1, bias1,
           W2, att_src2, att_dst2, bias2,
           bn1_gamma, bn1_beta, bn2_gamma, bn2_beta, fc_W, fc_b):
  src = edge_index[0].astype(jnp.int32)
  dst = edge_index[1].astype(jnp.int32)
  batch3 = batch.astype(jnp.int32).reshape(10, 1, 1000)

  Ws1 = W1.reshape(IN, HEADS, HID).transpose(1, 0, 2).reshape(HEADS * IN, HID)
  Ws2 = W2.reshape(HID, HEADS, HID).transpose(1, 0, 2).reshape(HEADS * HID, HID)
  b1 = bias1.reshape(1, HID)
  b2 = bias2.reshape(1, HID)
  g1 = bn1_gamma.reshape(1, HID)
  g2 = bn2_gamma.reshape(1, HID)
  be1 = bn1_beta.reshape(1, HID)
  be2 = bn2_beta.reshape(1, HID)
  fbr = fc_b.reshape(1, OUT)

  sad1 = _sad_call(x, W1, att_src1, att_dst1)
  agg1 = _sc_edge(sad1, src, dst, x)
  h1, sad2 = _dense_call(agg1.reshape(-1, HEADS * IN), Ws1, b1, g1, be1,
                         W2, att_src2, att_dst2)
  agg2 = _sc_edge(sad2, src, dst, h1)
  return _final_call(agg2.reshape(-1, HEADS * HID), Ws2, b2, g2, be2,
                     batch3, fc_W, fbr)
